# baseline (speedup 1.0000x reference)
import functools

import jax
import jax.numpy as jnp
from jax import lax
from jax.experimental import pallas as pl
from jax.experimental.pallas import tpu as pltpu
from jax.experimental.pallas import tpu_sc as plsc

NUM_CORES = 2
NUM_SUBCORES = 16
NUM_WORKERS = NUM_CORES * NUM_SUBCORES

STREAM = 128
STREAMS_PER_SUPER = 2
SUPER = STREAM * STREAMS_PER_SUPER


def _gather_body(per_w, n_super, d, table_hbm, idx_hbm, out_hbm,
                 idx_v, rows0, rows1, sem0, sem1):
    wid = lax.axis_index("s") * NUM_CORES + lax.axis_index("c")
    base = wid * per_w
    pltpu.sync_copy(idx_hbm.at[wid], idx_v)

    bufs = (rows0, rows1)
    sems = (sem0, sem1)

    def issue(sup):
        buf = bufs[sup % 2]
        sem = sems[sup % 2]
        cps = []
        for j in range(STREAMS_PER_SUPER):
            s = sup * STREAMS_PER_SUPER + j
            cps.append(pltpu.async_copy(
                table_hbm.at[idx_v.at[s]],
                buf.at[pl.ds(j * STREAM, STREAM)],
                sem))
        return cps

    pending = [issue(0), None]
    for sup in range(n_super):
        nxt = sup + 1
        if nxt < n_super:
            pending[nxt % 2] = issue(nxt)
        for cp in pending[sup % 2]:
            cp.wait()
        pltpu.sync_copy(bufs[sup % 2],
                        out_hbm.at[pl.ds(base + sup * SUPER, SUPER)])


def _sc_gather(table4, idx4):
    n = idx4.shape[0]
    d = table4.shape[1]
    per_w = n // NUM_WORKERS
    n_super = per_w // SUPER
    assert per_w % SUPER == 0
    mesh = plsc.VectorSubcoreMesh(core_axis_name="c", subcore_axis_name="s")
    f = pl.kernel(
        functools.partial(_gather_body, per_w, n_super, d),
        out_type=jax.ShapeDtypeStruct((n, d), jnp.float32),
        mesh=mesh,
        scratch_types=[
            pltpu.VMEM((per_w // STREAM, STREAM), jnp.int32),
            pltpu.VMEM((SUPER, d), jnp.float32),
            pltpu.VMEM((SUPER, d), jnp.float32),
            pltpu.SemaphoreType.DMA,
            pltpu.SemaphoreType.DMA,
        ],
    )
    return f(table4, idx4.reshape(NUM_WORKERS, per_w // STREAM, STREAM))


def _select_matmul_body(g_ref, pos_ref, w_ref, b_ref, out_ref):
    g = g_ref[...]
    quarter = lax.broadcasted_iota(jnp.int32, g.shape, 1) // 32
    mask = quarter == pos_ref[...]
    masked = jnp.where(mask, g, 0.0)
    out_ref[...] = jnp.dot(
        masked, w_ref[...], preferred_element_type=jnp.float32
    ) + b_ref[...]


def _tc_select_matmul(g, pos, Wstack, b):
    n, d4 = g.shape
    o = Wstack.shape[1]
    blk = 4096
    return pl.pallas_call(
        _select_matmul_body,
        grid=(n // blk,),
        in_specs=[
            pl.BlockSpec((blk, d4), lambda i: (i, 0)),
            pl.BlockSpec((blk, 1), lambda i: (i, 0)),
            pl.BlockSpec((d4, o), lambda i: (0, 0)),
            pl.BlockSpec((1, o), lambda i: (0, 0)),
        ],
        out_specs=pl.BlockSpec((blk, o), lambda i: (i, 0)),
        out_shape=jax.ShapeDtypeStruct((n, o), jnp.float32),
    )(g, pos, Wstack, b.reshape(1, o))


def kernel(x, table, W, b):
    bsz, seq = x.shape
    o = W.shape[1]
    xf = x.reshape(-1).astype(jnp.int32)
    table4 = table.reshape(table.shape[0] // 4, 4 * table.shape[1])
    g = _sc_gather(table4, xf >> 2)
    pos = (xf & 3).reshape(-1, 1)
    Wstack = jnp.concatenate([W, W, W, W], axis=0)
    out = _tc_select_matmul(g, pos, Wstack, b)
    return out.reshape(bsz, seq, o)


# speedup vs baseline: 1.3139x; 1.3139x over previous
import functools

import jax
import jax.numpy as jnp
from jax import lax
from jax.experimental import pallas as pl
from jax.experimental.pallas import tpu as pltpu
from jax.experimental.pallas import tpu_sc as plsc

NUM_CORES = 2
NUM_SUBCORES = 16
NUM_WORKERS = NUM_CORES * NUM_SUBCORES

STREAM = 128
STREAMS_PER_SUPER = 2
SUPER = STREAM * STREAMS_PER_SUPER


def _gather_body(per_w, n_super, d, table_hbm, idx_hbm, out_hbm,
                 idx_v, rows0, rows1, sem0, sem1):
    wid = lax.axis_index("s") * NUM_CORES + lax.axis_index("c")
    base = wid * per_w
    pltpu.sync_copy(idx_hbm.at[wid], idx_v)

    bufs = (rows0, rows1)
    sems = (sem0, sem1)

    def issue(sup):
        buf = bufs[sup % 2]
        sem = sems[sup % 2]
        cps = []
        for j in range(STREAMS_PER_SUPER):
            s = sup * STREAMS_PER_SUPER + j
            cps.append(pltpu.async_copy(
                table_hbm.at[idx_v.at[s]],
                buf.at[pl.ds(j * STREAM, STREAM)],
                sem))
        return cps

    pending = [issue(0), None]
    for sup in range(n_super):
        nxt = sup + 1
        if nxt < n_super:
            pending[nxt % 2] = issue(nxt)
        for cp in pending[sup % 2]:
            cp.wait()
        pltpu.sync_copy(bufs[sup % 2],
                        out_hbm.at[pl.ds(base + sup * SUPER, SUPER)])


def _sc_gather(table4, idx4):
    n = idx4.shape[0]
    d = table4.shape[1]
    per_w = n // NUM_WORKERS
    n_super = per_w // SUPER
    assert per_w % SUPER == 0
    mesh = plsc.VectorSubcoreMesh(core_axis_name="c", subcore_axis_name="s")
    f = pl.kernel(
        functools.partial(_gather_body, per_w, n_super, d),
        out_type=jax.ShapeDtypeStruct((n, d), jnp.float32),
        mesh=mesh,
        scratch_types=[
            pltpu.VMEM((per_w // STREAM, STREAM), jnp.int32),
            pltpu.VMEM((SUPER, d), jnp.float32),
            pltpu.VMEM((SUPER, d), jnp.float32),
            pltpu.SemaphoreType.DMA,
            pltpu.SemaphoreType.DMA,
        ],
    )
    return f(table4, idx4.reshape(NUM_WORKERS, per_w // STREAM, STREAM))


def _select_matmul_body(g_ref, x_ref, w_ref, b_ref, out_ref):
    g = g_ref[0]
    xv = x_ref[0]
    pos = jnp.reshape(xv & 3, (xv.shape[1], 1))
    quarter = lax.broadcasted_iota(jnp.int32, g.shape, 1) >> 5
    masked = jnp.where(quarter == pos, g, 0.0)
    m = jnp.dot(masked, w_ref[...], preferred_element_type=jnp.float32)
    out_ref[0] = jnp.transpose(m + b_ref[...])


def _tc_select_matmul(g3, xT3, Wstack, b):
    seq, bsz, d4 = g3.shape
    o = Wstack.shape[1]
    return pl.pallas_call(
        _select_matmul_body,
        grid=(seq,),
        in_specs=[
            pl.BlockSpec((1, bsz, d4), lambda i: (i, 0, 0)),
            pl.BlockSpec((1, 1, bsz), lambda i: (i, 0, 0)),
            pl.BlockSpec((d4, o), lambda i: (0, 0)),
            pl.BlockSpec((1, o), lambda i: (0, 0)),
        ],
        out_specs=pl.BlockSpec((1, o, bsz), lambda i: (i, 0, 0)),
        out_shape=jax.ShapeDtypeStruct((seq, o, bsz), jnp.float32),
    )(g3, xT3, Wstack, b.reshape(1, o))


def kernel(x, table, W, b):
    bsz, seq = x.shape
    o = W.shape[1]
    xT = x.T.astype(jnp.int32)
    xf = xT.reshape(-1)
    table4 = table.reshape(table.shape[0] // 4, 4 * table.shape[1])
    g = _sc_gather(table4, xf >> 2)
    g3 = g.reshape(seq, bsz, table4.shape[1])
    Wstack = jnp.concatenate([W, W, W, W], axis=0)
    outT = _tc_select_matmul(g3, xT.reshape(seq, 1, bsz), Wstack, b)
    return outT.transpose(2, 0, 1)
